# baseline (speedup 1.0000x reference)
import numpy as np
import jax
import jax.numpy as jnp
from jax import lax
from jax.experimental import pallas as pl
from jax.experimental.pallas import tpu as pltpu

_T = 20
_KMAX = 64
_RMAX = 48
_BPIX = 2048
_TC = 4

f32 = jnp.float32
i32 = jnp.int32


def _tf2x32_host(k0, k1, x0, x1):
    k0 = np.uint32(k0); k1 = np.uint32(k1)
    x0 = np.uint32(x0); x1 = np.uint32(x1)
    ks = [k0, k1, np.uint32(k0 ^ k1 ^ np.uint32(0x1BD11BDA))]
    rot = [(13, 15, 26, 6), (17, 29, 16, 24)]
    x0 = np.uint32(x0 + ks[0]); x1 = np.uint32(x1 + ks[1])
    for g in range(5):
        for r in rot[g % 2]:
            x0 = np.uint32(x0 + x1)
            x1 = np.uint32((np.uint32(x1 << np.uint32(r)) | np.uint32(x1 >> np.uint32(32 - r))))
            x1 = np.uint32(x0 ^ x1)
        x0 = np.uint32(x0 + ks[(g + 1) % 3])
        x1 = np.uint32(x1 + ks[(g + 2) % 3] + np.uint32(g + 1))
    return x0, x1


def _knuth_subkeys(n):
    k0, k1 = np.uint32(0), np.uint32(42)
    out = np.zeros((n, 2), np.uint32)
    for i in range(n):
        nk = _tf2x32_host(k0, k1, 0, 0)
        sk = _tf2x32_host(k0, k1, 0, 1)
        out[i] = sk
        k0, k1 = nk
    return out.view(np.int32)


def _rej_subkeys(n):
    k0, k1 = np.uint32(0), np.uint32(42)
    out = np.zeros((n, 4), np.uint32)
    for i in range(n):
        nk = _tf2x32_host(k0, k1, 0, 0)
        a = _tf2x32_host(k0, k1, 0, 1)
        b = _tf2x32_host(k0, k1, 0, 2)
        out[i, 0], out[i, 1] = a
        out[i, 2], out[i, 3] = b
        k0, k1 = nk
    return out.view(np.int32)


_KSUB = _knuth_subkeys(_KMAX)
_RSUB = _rej_subkeys(_RMAX)

_LANCZOS = [676.520368121885098567009190444019,
            -1259.13921672240287047156078755283,
            771.3234287776530788486528258894,
            -176.61502916214059906584551354,
            12.507343278686904814458936853,
            -0.13857109526572011689554707,
            9.984369578019570859563e-6,
            1.50563273514931155834e-7]
_BASE_LANCZOS = 0.99999999999980993227684700473478
_LOG_SQRT_2PI = 0.91893853320467274178032973640562
_LOG_7P5 = float(np.log(np.float64(7.5)))


def _rotl(x, r):
    return lax.shift_left(x, i32(r)) | lax.shift_right_logical(x, i32(32 - r))


def _threefry(k0, k1, counts):
    ks2 = k0 ^ k1 ^ i32(0x1BD11BDA)
    ks = (k0, k1, ks2)
    rot = ((13, 15, 26, 6), (17, 29, 16, 24))
    x0 = jnp.zeros_like(counts) + k0
    x1 = counts + k1
    for g in range(5):
        for r in rot[g % 2]:
            x0 = x0 + x1
            x1 = _rotl(x1, r)
            x1 = x0 ^ x1
        x0 = x0 + ks[(g + 1) % 3]
        x1 = x1 + ks[(g + 2) % 3] + i32(g + 1)
    return x0 ^ x1


def _uniform(k0, k1, counts):
    bits = _threefry(k0, k1, counts)
    fb = lax.shift_right_logical(bits, i32(9)) | i32(0x3F800000)
    return lax.bitcast_convert_type(fb, f32) - f32(1.0)


def _lgamma(x):
    z = x - f32(1.0)
    s = jnp.full_like(x, f32(_BASE_LANCZOS))
    for idx, c in enumerate(_LANCZOS):
        s = s + f32(c) / (z + f32(idx + 1))
    t = z + f32(7.5)
    log_t = f32(_LOG_7P5) + jnp.log1p(z / f32(7.5))
    return f32(_LOG_SQRT_2PI) + (z + f32(0.5) - t / log_t) * log_t + jnp.log(s)


def _lam_of(x):
    nz = x != f32(0.0)
    lam = jnp.where(nz, f32(1.0) / jnp.where(nz, x, f32(1.0)), f32(0.0))
    return nz, lam


def _rej_consts(lam_rej):
    log_lam = jnp.log(lam_rej)
    b_ = f32(0.931) + f32(2.53) * jnp.sqrt(lam_rej)
    a_ = f32(-0.059) + f32(0.02483) * b_
    inv_alpha = f32(1.1239) + f32(1.1328) / (b_ - f32(3.4))
    v_r = f32(0.9277) - f32(3.6224) / (b_ - f32(2.0))
    return log_lam, b_, a_, inv_alpha, v_r


def _rej_iter(s0a, s0b, s1a, s1b, counts, lam_rej, consts):
    log_lam, b_, a_, inv_alpha, v_r = consts
    u = _uniform(s0a, s0b, counts) - f32(0.5)
    v = _uniform(s1a, s1b, counts)
    u_sh = f32(0.5) - jnp.abs(u)
    k = jnp.floor((f32(2.0) * a_ / u_sh + b_) * u + lam_rej + f32(0.43))
    s = jnp.log(v * inv_alpha / (a_ / (u_sh * u_sh) + b_))
    t = -lam_rej + k * log_lam - _lgamma(k + f32(1.0))
    accept1 = (u_sh >= f32(0.07)) & (v <= v_r)
    reject = (k < f32(0.0)) | ((u_sh < f32(0.013)) & (v > u_sh))
    accept2 = s <= t
    return accept1 | ((~reject) & accept2), k


def _phase_a_body(n_total, img_ref, idx_ref, rsub_ref, itot_ref):
    b = pl.program_id(0)
    x = img_ref[0]
    _, lam = _lam_of(x)
    use_kn = lam < f32(10.0)
    lam_rej = jnp.where(use_kn, f32(1e5), lam)
    consts = _rej_consts(lam_rej)

    n_glob = idx_ref[0]
    pad_i = (n_glob >= n_total).astype(i32)

    cshape = (_TC, 8, 256)
    t_iota = lax.broadcasted_iota(i32, cshape, 0) * i32(n_total)
    fa_init = jnp.where(jnp.broadcast_to(pad_i[None], cshape) == 1,
                        i32(0), i32(_RMAX))
    iters = i32(0)
    for t0 in range(0, _T, _TC):
        counts = t_iota + (n_glob + i32(t0 * n_total))[None]

        def cond(c):
            i, fa = c
            return (i < _RMAX) & jnp.any(fa == _RMAX)

        def body(c):
            i, fa = c
            for j in range(2):
                accept, _ = _rej_iter(rsub_ref[i + j, 0], rsub_ref[i + j, 1],
                                      rsub_ref[i + j, 2], rsub_ref[i + j, 3],
                                      counts, lam_rej, consts)
                fa = jnp.where(accept & (fa == _RMAX), i + j, fa)
            return i + 2, fa

        _, fa = lax.while_loop(cond, body, (i32(0), fa_init))
        iters = jnp.maximum(iters, jnp.max(fa) + 1)

    @pl.when(b == 0)
    def _():
        itot_ref[...] = jnp.zeros((8, 128), i32)

    itot_ref[...] = jnp.maximum(itot_ref[...], iters)


def _main_body(n_total, img_ref, idx_ref, ksub_ref, rsub_ref, itot_ref, out_ref):
    b = pl.program_id(0)
    x = img_ref[0]
    nz, lam = _lam_of(x)
    use_kn = lam < f32(10.0)
    lam_kn = jnp.where(use_kn, lam, f32(0.0))
    lam_rej = jnp.where(use_kn, f32(1e5), lam)
    consts = _rej_consts(lam_rej)

    n_glob = idx_ref[0]
    pad = (n_glob >= n_total)

    neg_lam = -lam_kn
    i_total = jnp.max(itot_ref[...])
    found_init = (use_kn | pad).astype(i32)
    zero_lam = lam == f32(0.0)

    cshape = (_TC, 8, 256)
    t_iota = lax.broadcasted_iota(i32, cshape, 0) * i32(n_total)
    found_init_c = jnp.broadcast_to(found_init[None], cshape)
    acc = jnp.zeros((8, 256), f32)
    rows = []
    for t0 in range(0, _T, _TC):
        counts = t_iota + (n_glob + i32(t0 * n_total))[None]

        def kn_cond(c):
            i, k, lp = c
            return (i < _KMAX) & jnp.any(lp > neg_lam)

        def kn_body(c):
            i, k, lp = c
            for j in range(4):
                k = jnp.where(lp > neg_lam, k + 1, k)
                u = _uniform(ksub_ref[i + j, 0], ksub_ref[i + j, 1], counts)
                lp = lp + jnp.log(u)
            return i + 4, k, lp

        _, kk, _ = lax.while_loop(
            kn_cond, kn_body,
            (i32(0), jnp.zeros(cshape, i32), jnp.zeros(cshape, f32)))
        kn_res = kk - 1

        def rj_cond(c):
            i, found, kout = c
            return (i >= 0) & jnp.any(found == 0)

        def rj_body(c):
            i, found, kout = c
            accept, k = _rej_iter(rsub_ref[i, 0], rsub_ref[i, 1],
                                  rsub_ref[i, 2], rsub_ref[i, 3],
                                  counts, lam_rej, consts)
            newly = accept & (found == 0)
            return i - 1, found | accept.astype(i32), jnp.where(newly, k, kout)

        _, _, kout = lax.while_loop(
            rj_cond, rj_body,
            (i_total - 1, found_init_c, jnp.full(cshape, f32(-1.0))))
        rej_res = kout.astype(i32)

        res = jnp.where(use_kn[None], kn_res, rej_res)
        res = jnp.where(zero_lam[None], 0, res)
        iv = res.astype(f32)
        iv = jnp.where(nz[None] & (iv == f32(0.0)), f32(1.0), iv)
        for j in range(_TC):
            acc = acc + iv[j]
            tf = jnp.where(acc >= f32(_T + 1), f32(0.0), acc)
            rows.append(tf.astype(i32))

    for s in range(_T):
        val = jnp.zeros((8, 256), i32)
        for t in range(_T):
            val = val | jnp.where(rows[t] == s + 1, 1, 0)
        out_ref[0, 8 * s:8 * (s + 1), :] = val.astype(jnp.int8)


def kernel(img):
    orig_shape = img.shape
    n_total = img.size
    nblk = (n_total + _BPIX - 1) // _BPIX
    npad = nblk * _BPIX - n_total

    flat = img.reshape(-1).astype(f32)

    nzf = flat != f32(0.0)
    lam_f = jnp.where(nzf, f32(1.0) / jnp.where(nzf, flat, f32(1.0)), f32(0.0))
    bucket = (jnp.where(lam_f >= f32(2.0), 1, 0)
              + jnp.where(lam_f >= f32(4.0), 1, 0)
              + jnp.where(lam_f >= f32(7.0), 1, 0)
              + jnp.where(lam_f >= f32(10.0), 1, 0))
    pos = jnp.zeros((n_total,), i32)
    offset = i32(0)
    for bk in range(5):
        ind = (bucket == bk).astype(i32)
        rank = jnp.cumsum(ind) - 1
        pos = jnp.where(ind == 1, offset + rank, pos)
        offset = offset + rank[-1] + 1
    iota_n = jnp.arange(n_total, dtype=i32)
    perm = jnp.zeros((n_total,), i32).at[pos].set(iota_n)

    flat_p = jnp.take(flat, perm)
    idx_p = perm
    if npad:
        flat_p = jnp.concatenate([flat_p, jnp.zeros((npad,), f32)])
        idx_p = jnp.concatenate(
            [idx_p, jnp.full((npad,), n_total, i32)])
    blocks = flat_p.reshape(nblk, 8, 256)
    idx_blocks = idx_p.reshape(nblk, 8, 256)

    ksub = jnp.asarray(_KSUB)
    rsub = jnp.asarray(_RSUB)

    itot = pl.pallas_call(
        lambda *a: _phase_a_body(n_total, *a),
        grid=(nblk,),
        in_specs=[
            pl.BlockSpec((1, 8, 256), lambda b: (b, 0, 0)),
            pl.BlockSpec((1, 8, 256), lambda b: (b, 0, 0)),
            pl.BlockSpec(memory_space=pltpu.SMEM),
        ],
        out_specs=pl.BlockSpec((8, 128), lambda b: (0, 0)),
        out_shape=jax.ShapeDtypeStruct((8, 128), i32),
        compiler_params=pltpu.CompilerParams(
            dimension_semantics=("arbitrary",)),
    )(blocks, idx_blocks, rsub)

    spikes = pl.pallas_call(
        lambda *a: _main_body(n_total, *a),
        grid=(nblk,),
        in_specs=[
            pl.BlockSpec((1, 8, 256), lambda b: (b, 0, 0)),
            pl.BlockSpec((1, 8, 256), lambda b: (b, 0, 0)),
            pl.BlockSpec(memory_space=pltpu.SMEM),
            pl.BlockSpec(memory_space=pltpu.SMEM),
            pl.BlockSpec((8, 128), lambda b: (0, 0)),
        ],
        out_specs=pl.BlockSpec((1, _T * 8, 256), lambda b: (b, 0, 0)),
        out_shape=jax.ShapeDtypeStruct((nblk, _T * 8, 256), jnp.int8),
        compiler_params=pltpu.CompilerParams(
            dimension_semantics=("parallel",)),
    )(blocks, idx_blocks, ksub, rsub, itot)

    sp = spikes.reshape(nblk, _T, 8, 256).transpose(1, 0, 2, 3).reshape(_T, -1)
    if npad:
        sp = sp[:, :n_total]
    sp = jnp.take(sp, pos, axis=1)
    return sp.astype(jnp.bool_).reshape((_T,) + tuple(orig_shape))


# speedup vs baseline: 1.8712x; 1.8712x over previous
import numpy as np
import jax
import jax.numpy as jnp
from jax import lax
from jax.experimental import pallas as pl
from jax.experimental.pallas import tpu as pltpu

_T = 20
_KMAX = 64
_RMAX = 48
_BPIX = 2048
_TC = 4

f32 = jnp.float32
i32 = jnp.int32


def _tf2x32_host(k0, k1, x0, x1):
    k0 = np.uint32(k0); k1 = np.uint32(k1)
    x0 = np.uint32(x0); x1 = np.uint32(x1)
    ks = [k0, k1, np.uint32(k0 ^ k1 ^ np.uint32(0x1BD11BDA))]
    rot = [(13, 15, 26, 6), (17, 29, 16, 24)]
    x0 = np.uint32(x0 + ks[0]); x1 = np.uint32(x1 + ks[1])
    for g in range(5):
        for r in rot[g % 2]:
            x0 = np.uint32(x0 + x1)
            x1 = np.uint32((np.uint32(x1 << np.uint32(r)) | np.uint32(x1 >> np.uint32(32 - r))))
            x1 = np.uint32(x0 ^ x1)
        x0 = np.uint32(x0 + ks[(g + 1) % 3])
        x1 = np.uint32(x1 + ks[(g + 2) % 3] + np.uint32(g + 1))
    return x0, x1


def _knuth_subkeys(n):
    k0, k1 = np.uint32(0), np.uint32(42)
    out = np.zeros((n, 2), np.uint32)
    for i in range(n):
        nk = _tf2x32_host(k0, k1, 0, 0)
        sk = _tf2x32_host(k0, k1, 0, 1)
        out[i] = sk
        k0, k1 = nk
    return out.view(np.int32)


def _rej_subkeys(n):
    k0, k1 = np.uint32(0), np.uint32(42)
    out = np.zeros((n, 4), np.uint32)
    for i in range(n):
        nk = _tf2x32_host(k0, k1, 0, 0)
        a = _tf2x32_host(k0, k1, 0, 1)
        b = _tf2x32_host(k0, k1, 0, 2)
        out[i, 0], out[i, 1] = a
        out[i, 2], out[i, 3] = b
        k0, k1 = nk
    return out.view(np.int32)


_KSUB = _knuth_subkeys(_KMAX)
_RSUB = _rej_subkeys(_RMAX)

_LANCZOS = [676.520368121885098567009190444019,
            -1259.13921672240287047156078755283,
            771.3234287776530788486528258894,
            -176.61502916214059906584551354,
            12.507343278686904814458936853,
            -0.13857109526572011689554707,
            9.984369578019570859563e-6,
            1.50563273514931155834e-7]
_BASE_LANCZOS = 0.99999999999980993227684700473478
_LOG_SQRT_2PI = 0.91893853320467274178032973640562
_LOG_7P5 = float(np.log(np.float64(7.5)))


def _rotl(x, r):
    return lax.shift_left(x, i32(r)) | lax.shift_right_logical(x, i32(32 - r))


def _threefry(k0, k1, counts):
    ks2 = k0 ^ k1 ^ i32(0x1BD11BDA)
    ks = (k0, k1, ks2)
    rot = ((13, 15, 26, 6), (17, 29, 16, 24))
    x0 = jnp.zeros_like(counts) + k0
    x1 = counts + k1
    for g in range(5):
        for r in rot[g % 2]:
            x0 = x0 + x1
            x1 = _rotl(x1, r)
            x1 = x0 ^ x1
        x0 = x0 + ks[(g + 1) % 3]
        x1 = x1 + ks[(g + 2) % 3] + i32(g + 1)
    return x0 ^ x1


def _uniform(k0, k1, counts):
    bits = _threefry(k0, k1, counts)
    fb = lax.shift_right_logical(bits, i32(9)) | i32(0x3F800000)
    return lax.bitcast_convert_type(fb, f32) - f32(1.0)


def _lgamma(x):
    z = x - f32(1.0)
    s = jnp.full_like(x, f32(_BASE_LANCZOS))
    for idx, c in enumerate(_LANCZOS):
        s = s + f32(c) / (z + f32(idx + 1))
    t = z + f32(7.5)
    log_t = f32(_LOG_7P5) + jnp.log1p(z / f32(7.5))
    return f32(_LOG_SQRT_2PI) + (z + f32(0.5) - t / log_t) * log_t + jnp.log(s)


def _lam_of(x):
    nz = x != f32(0.0)
    lam = jnp.where(nz, f32(1.0) / jnp.where(nz, x, f32(1.0)), f32(0.0))
    return nz, lam


def _rej_consts(lam_rej):
    log_lam = jnp.log(lam_rej)
    b_ = f32(0.931) + f32(2.53) * jnp.sqrt(lam_rej)
    a_ = f32(-0.059) + f32(0.02483) * b_
    inv_alpha = f32(1.1239) + f32(1.1328) / (b_ - f32(3.4))
    v_r = f32(0.9277) - f32(3.6224) / (b_ - f32(2.0))
    return log_lam, b_, a_, inv_alpha, v_r


def _rej_iter(s0a, s0b, s1a, s1b, counts, lam_rej, consts):
    log_lam, b_, a_, inv_alpha, v_r = consts
    u = _uniform(s0a, s0b, counts) - f32(0.5)
    v = _uniform(s1a, s1b, counts)
    u_sh = f32(0.5) - jnp.abs(u)
    k = jnp.floor((f32(2.0) * a_ / u_sh + b_) * u + lam_rej + f32(0.43))
    s = jnp.log(v * inv_alpha / (a_ / (u_sh * u_sh) + b_))
    t = -lam_rej + k * log_lam - _lgamma(k + f32(1.0))
    accept1 = (u_sh >= f32(0.07)) & (v <= v_r)
    reject = (k < f32(0.0)) | ((u_sh < f32(0.013)) & (v > u_sh))
    accept2 = s <= t
    return accept1 | ((~reject) & accept2), k


def _phase_a_body(n_total, img_ref, idx_ref, rsub_ref, itot_ref):
    b = pl.program_id(0)
    x = img_ref[0]
    _, lam = _lam_of(x)
    use_kn = lam < f32(10.0)
    lam_rej = jnp.where(use_kn, f32(1e5), lam)
    consts = _rej_consts(lam_rej)

    n_glob = idx_ref[0]
    pad_i = (n_glob >= n_total).astype(i32)

    cshape = (_TC, 8, 256)
    t_iota = lax.broadcasted_iota(i32, cshape, 0) * i32(n_total)
    fa_init = jnp.where(jnp.broadcast_to(pad_i[None], cshape) == 1,
                        i32(0), i32(_RMAX))
    iters = i32(0)
    for t0 in range(0, _T, _TC):
        counts = t_iota + (n_glob + i32(t0 * n_total))[None]

        def cond(c):
            i, fa = c
            return (i < _RMAX) & jnp.any(fa == _RMAX)

        def body(c):
            i, fa = c
            for j in range(2):
                accept, _ = _rej_iter(rsub_ref[i + j, 0], rsub_ref[i + j, 1],
                                      rsub_ref[i + j, 2], rsub_ref[i + j, 3],
                                      counts, lam_rej, consts)
                fa = jnp.where(accept & (fa == _RMAX), i + j, fa)
            return i + 2, fa

        _, fa = lax.while_loop(cond, body, (i32(0), fa_init))
        iters = jnp.maximum(iters, jnp.max(fa) + 1)

    @pl.when(b == 0)
    def _():
        itot_ref[...] = jnp.zeros((8, 128), i32)

    itot_ref[...] = jnp.maximum(itot_ref[...], iters)


def _main_body(n_total, img_ref, idx_ref, ksub_ref, rsub_ref, itot_ref, out_ref):
    b = pl.program_id(0)
    x = img_ref[0]
    nz, lam = _lam_of(x)
    use_kn = lam < f32(10.0)
    lam_kn = jnp.where(use_kn, lam, f32(0.0))
    lam_rej = jnp.where(use_kn, f32(1e5), lam)
    consts = _rej_consts(lam_rej)

    n_glob = idx_ref[0]
    pad = (n_glob >= n_total)

    neg_lam = -lam_kn
    i_total = jnp.max(itot_ref[...])
    found_init = (use_kn | pad).astype(i32)
    zero_lam = lam == f32(0.0)

    cshape = (_TC, 8, 256)
    t_iota = lax.broadcasted_iota(i32, cshape, 0) * i32(n_total)
    found_init_c = jnp.broadcast_to(found_init[None], cshape)
    acc = jnp.zeros((8, 256), f32)
    rows = []
    for t0 in range(0, _T, _TC):
        counts = t_iota + (n_glob + i32(t0 * n_total))[None]

        def kn_cond(c):
            i, k, lp = c
            return (i < _KMAX) & jnp.any(lp > neg_lam)

        def kn_body(c):
            i, k, lp = c
            for j in range(4):
                k = jnp.where(lp > neg_lam, k + 1, k)
                u = _uniform(ksub_ref[i + j, 0], ksub_ref[i + j, 1], counts)
                lp = lp + jnp.log(u)
            return i + 4, k, lp

        _, kk, _ = lax.while_loop(
            kn_cond, kn_body,
            (i32(0), jnp.zeros(cshape, i32), jnp.zeros(cshape, f32)))
        kn_res = kk - 1

        def rj_cond(c):
            i, found, kout = c
            return (i >= 0) & jnp.any(found == 0)

        def rj_body(c):
            i, found, kout = c
            accept, k = _rej_iter(rsub_ref[i, 0], rsub_ref[i, 1],
                                  rsub_ref[i, 2], rsub_ref[i, 3],
                                  counts, lam_rej, consts)
            newly = accept & (found == 0)
            return i - 1, found | accept.astype(i32), jnp.where(newly, k, kout)

        _, _, kout = lax.while_loop(
            rj_cond, rj_body,
            (i_total - 1, found_init_c, jnp.full(cshape, f32(-1.0))))
        rej_res = kout.astype(i32)

        res = jnp.where(use_kn[None], kn_res, rej_res)
        res = jnp.where(zero_lam[None], 0, res)
        iv = res.astype(f32)
        iv = jnp.where(nz[None] & (iv == f32(0.0)), f32(1.0), iv)
        for j in range(_TC):
            acc = acc + iv[j]
            tf = jnp.where(acc >= f32(_T + 1), f32(0.0), acc)
            rows.append(tf.astype(i32))

    bm = jnp.zeros((8, 256), i32)
    for t in range(_T):
        tf = rows[t]
        sh = jnp.where(tf > 0, tf - 1, 0)
        bm = bm | jnp.where(tf > 0, jnp.left_shift(i32(1), sh), 0)
    out_ref[0] = bm


def _expand_body(bm_ref, out_ref):
    bm = bm_ref[0]
    for st in range(_T):
        out_ref[0, 8 * st:8 * (st + 1), :] = (
            lax.shift_right_logical(bm, i32(st)) & 1).astype(jnp.int8)


def kernel(img):
    orig_shape = img.shape
    n_total = img.size
    nblk = (n_total + _BPIX - 1) // _BPIX
    npad = nblk * _BPIX - n_total

    flat = img.reshape(-1).astype(f32)

    nzf = flat != f32(0.0)
    lam_f = jnp.where(nzf, f32(1.0) / jnp.where(nzf, flat, f32(1.0)), f32(0.0))
    bucket = (jnp.where(lam_f >= f32(2.0), 1, 0)
              + jnp.where(lam_f >= f32(4.0), 1, 0)
              + jnp.where(lam_f >= f32(7.0), 1, 0)
              + jnp.where(lam_f >= f32(10.0), 1, 0))
    pos = jnp.zeros((n_total,), i32)
    offset = i32(0)
    for bk in range(5):
        ind = (bucket == bk).astype(i32)
        rank = jnp.cumsum(ind) - 1
        pos = jnp.where(ind == 1, offset + rank, pos)
        offset = offset + rank[-1] + 1
    iota_n = jnp.arange(n_total, dtype=i32)
    perm = jnp.zeros((n_total,), i32).at[pos].set(iota_n)

    flat_p = jnp.take(flat, perm)
    idx_p = perm
    if npad:
        flat_p = jnp.concatenate([flat_p, jnp.zeros((npad,), f32)])
        idx_p = jnp.concatenate(
            [idx_p, jnp.full((npad,), n_total, i32)])
    blocks = flat_p.reshape(nblk, 8, 256)
    idx_blocks = idx_p.reshape(nblk, 8, 256)

    ksub = jnp.asarray(_KSUB)
    rsub = jnp.asarray(_RSUB)

    itot = pl.pallas_call(
        lambda *a: _phase_a_body(n_total, *a),
        grid=(nblk,),
        in_specs=[
            pl.BlockSpec((1, 8, 256), lambda b: (b, 0, 0)),
            pl.BlockSpec((1, 8, 256), lambda b: (b, 0, 0)),
            pl.BlockSpec(memory_space=pltpu.SMEM),
        ],
        out_specs=pl.BlockSpec((8, 128), lambda b: (0, 0)),
        out_shape=jax.ShapeDtypeStruct((8, 128), i32),
        compiler_params=pltpu.CompilerParams(
            dimension_semantics=("arbitrary",)),
    )(blocks, idx_blocks, rsub)

    spikes = pl.pallas_call(
        lambda *a: _main_body(n_total, *a),
        grid=(nblk,),
        in_specs=[
            pl.BlockSpec((1, 8, 256), lambda b: (b, 0, 0)),
            pl.BlockSpec((1, 8, 256), lambda b: (b, 0, 0)),
            pl.BlockSpec(memory_space=pltpu.SMEM),
            pl.BlockSpec(memory_space=pltpu.SMEM),
            pl.BlockSpec((8, 128), lambda b: (0, 0)),
        ],
        out_specs=pl.BlockSpec((1, 8, 256), lambda b: (b, 0, 0)),
        out_shape=jax.ShapeDtypeStruct((nblk, 8, 256), i32),
        compiler_params=pltpu.CompilerParams(
            dimension_semantics=("parallel",)),
    )(blocks, idx_blocks, ksub, rsub, itot)

    bm_flat = spikes.reshape(-1)
    if npad:
        bm_flat = bm_flat[:n_total]
    bm_orig = jnp.take(bm_flat, pos)
    if npad:
        bm_orig = jnp.concatenate([bm_orig, jnp.zeros((npad,), i32)])

    sp = pl.pallas_call(
        _expand_body,
        grid=(nblk,),
        in_specs=[pl.BlockSpec((1, 8, 256), lambda b: (b, 0, 0))],
        out_specs=pl.BlockSpec((1, _T * 8, 256), lambda b: (b, 0, 0)),
        out_shape=jax.ShapeDtypeStruct((nblk, _T * 8, 256), jnp.int8),
        compiler_params=pltpu.CompilerParams(
            dimension_semantics=("parallel",)),
    )(bm_orig.reshape(nblk, 8, 256))

    sp = sp.reshape(nblk, _T, 8, 256).transpose(1, 0, 2, 3).reshape(_T, -1)
    if npad:
        sp = sp[:, :n_total]
    return sp.astype(jnp.bool_).reshape((_T,) + tuple(orig_shape))
